# baseline (speedup 1.0000x reference)
import functools

import jax
import jax.numpy as jnp
from jax import lax
from jax.experimental import pallas as pl
from jax.experimental.pallas import tpu as pltpu
from jax.experimental.pallas import tpu_sc as plsc

B = 4096
TWO_N = 128
N = 64
OUT_W = TWO_N * N + N + 2
FB = 512
P_PER = FB // N
N_MAIN = TWO_N * N // FB

TAIL = N + 2
NC = 2
NS = 16
NW = NC * NS
BPW = B // NW
L = 16


def _sc_tail_body(cardflat_hbm, flip_hbm, valid_hbm, t_hbm, out_hbm,
                  idx_v, fc_v, valid_v, t_v, tail_v, sem):
    wid = lax.axis_index("s") * NC + lax.axis_index("c")
    base = wid * BPW
    pltpu.sync_copy(flip_hbm.at[pl.ds(base, BPW)], idx_v)
    pltpu.sync_copy(valid_hbm.at[pl.ds(base, BPW)], valid_v)
    pltpu.sync_copy(t_hbm.at[pl.ds(base, BPW)], t_v)
    for k in range(BPW // L):
        sl = pl.ds(k * L, L)
        b = base + k * L + lax.broadcasted_iota(jnp.int32, (L,), 0)
        idx_v[sl] = b * TWO_N + idx_v[sl]
    pltpu.async_copy(cardflat_hbm.at[idx_v], fc_v, sem).wait()
    for k in range(BPW // L):
        sl = pl.ds(k * L, L)
        fc = fc_v[sl]
        va = valid_v[sl]
        for c in range(N):
            tail_v[c, sl] = jnp.where(fc == c, va, 0.0)
        par = jnp.bitwise_and(t_v[sl], 1)
        tail_v[N, sl] = jnp.where(par == 0, 1.0, 0.0)
        tail_v[N + 1, sl] = jnp.where(par == 1, 1.0, 0.0)
    pltpu.sync_copy(tail_v, out_hbm.at[:, wid])


_sc_tail = functools.partial(
    pl.kernel,
    mesh=plsc.VectorSubcoreMesh(core_axis_name="c", subcore_axis_name="s"),
    out_type=jax.ShapeDtypeStruct((TAIL, NW, BPW), jnp.float32),
    scratch_types=[
        pltpu.VMEM((BPW,), jnp.int32),
        pltpu.VMEM((BPW,), jnp.int32),
        pltpu.VMEM((BPW,), jnp.float32),
        pltpu.VMEM((BPW,), jnp.int32),
        pltpu.VMEM((TAIL, BPW), jnp.float32),
        pltpu.SemaphoreType.DMA,
    ],
)(_sc_tail_body)


def _tc_main_body(cardm_ref, out_ref):
    cm = cardm_ref[...]
    sub = jax.lax.broadcasted_iota(jnp.int32, (N, B), 0)
    for j in range(P_PER):
        row = jnp.broadcast_to(cm[j : j + 1, :], (N, B))
        out_ref[N * j : N * (j + 1), :] = jnp.where(row == sub, 1.0, 0.0)


def _tc_tail_body(src_ref, tail_ref, out_ref, sem):
    del src_ref
    copy = pltpu.make_async_copy(tail_ref, out_ref.at[pl.ds(TWO_N * N, TAIL)], sem)
    copy.start()
    copy.wait()


def kernel(card, seen_mask, flipped, flipped_valid, t, W):
    del W
    card32 = card.astype(jnp.int32)
    tail = _sc_tail(
        card32.reshape(B * TWO_N),
        flipped.astype(jnp.int32),
        flipped_valid.astype(jnp.float32),
        t.astype(jnp.int32),
    ).reshape(TAIL, 1, B)

    cardT = card32.T
    cardmT = jnp.where(seen_mask.T, cardT, 64)

    main = pl.pallas_call(
        _tc_main_body,
        grid=(N_MAIN,),
        in_specs=[pl.BlockSpec((P_PER, B), lambda i: (i, 0))],
        out_specs=pl.BlockSpec((FB, None, B), lambda i: (i, 0, 0)),
        out_shape=jax.ShapeDtypeStruct((OUT_W, 1, B), jnp.float32),
    )(cardmT)

    out = pl.pallas_call(
        _tc_tail_body,
        in_specs=[
            pl.BlockSpec(memory_space=pl.ANY),
            pl.BlockSpec(memory_space=pl.ANY),
        ],
        out_specs=pl.BlockSpec(memory_space=pl.ANY),
        out_shape=jax.ShapeDtypeStruct((OUT_W, 1, B), jnp.float32),
        scratch_shapes=[pltpu.SemaphoreType.DMA],
        input_output_aliases={0: 0},
    )(main, tail)
    return jnp.transpose(out, (2, 1, 0))


# speedup vs baseline: 1.8704x; 1.8704x over previous
import jax
import jax.numpy as jnp
from jax.experimental import pallas as pl

B = 4096
TWO_N = 128
N = 64
OUT_W = TWO_N * N + N + 2
FB = 512
P_PER = FB // N
N_MAIN = TWO_N * N // FB


def _body(cardm_ref, card_full_ref, flip_ref, valid_ref, t_ref, out_ref):
    i = pl.program_id(0)

    @pl.when(i < N_MAIN)
    def _main():
        cm = cardm_ref[...]
        sub = jax.lax.broadcasted_iota(jnp.int32, (N, B), 0)
        for j in range(P_PER):
            row = jnp.broadcast_to(cm[j : j + 1, :], (N, B))
            out_ref[N * j : N * (j + 1), :] = jnp.where(row == sub, 1.0, 0.0)

    @pl.when(i == N_MAIN)
    def _tail():
        card = card_full_ref[...]
        flip = flip_ref[...]
        prow = jax.lax.broadcasted_iota(jnp.int32, (TWO_N, B), 0)
        fcv = jnp.where(prow == flip, card, 0)
        fc = jnp.sum(fcv, axis=0, keepdims=True)
        sub = jax.lax.broadcasted_iota(jnp.int32, (N, B), 0)
        valid = valid_ref[...]
        out_ref[0:N, :] = jnp.where(sub == fc, valid, 0.0)
        par = jnp.bitwise_and(t_ref[...], 1)
        sub2 = jax.lax.broadcasted_iota(jnp.int32, (2, B), 0)
        out_ref[N : N + 2, :] = jnp.where(sub2 == par, 1.0, 0.0)


def kernel(card, seen_mask, flipped, flipped_valid, t, W):
    del W
    card32 = card.astype(jnp.int32)
    cardmT = jnp.where(seen_mask, card32, 64).T
    cardT = card32.T
    flipT = flipped.astype(jnp.int32).reshape(1, B)
    validT = flipped_valid.astype(jnp.float32).reshape(1, B)
    tT = t.astype(jnp.int32).reshape(1, B)

    grid = (N_MAIN + 1,)
    out = pl.pallas_call(
        _body,
        grid=grid,
        in_specs=[
            pl.BlockSpec((P_PER, B), lambda i: (jnp.minimum(i, N_MAIN - 1), 0)),
            pl.BlockSpec((TWO_N, B), lambda i: (0, 0)),
            pl.BlockSpec((1, B), lambda i: (0, 0)),
            pl.BlockSpec((1, B), lambda i: (0, 0)),
            pl.BlockSpec((1, B), lambda i: (0, 0)),
        ],
        out_specs=pl.BlockSpec((FB, None, B), lambda i: (i, 0, 0)),
        out_shape=jax.ShapeDtypeStruct((OUT_W, 1, B), jnp.float32),
    )(cardmT, cardT, flipT, validT, tT)
    return jnp.transpose(out, (2, 1, 0))


# speedup vs baseline: 1.8716x; 1.0006x over previous
import jax
import jax.numpy as jnp
from jax.experimental import pallas as pl

B = 4096
TWO_N = 128
N = 64
OUT_W = TWO_N * N + N + 2
FB = 512
P_PER = FB // N
N_MAIN = TWO_N * N // FB


def _body(cardm_ref, card_full_ref, flip_ref, valid_ref, t_ref, out_ref):
    i = pl.program_id(0)

    @pl.when(i < N_MAIN)
    def _main():
        cm = cardm_ref[...]
        sub8 = jax.lax.broadcasted_iota(jnp.int32, (8, B), 0)
        for j in range(P_PER):
            row8 = jnp.broadcast_to(cm[j : j + 1, :], (8, B))
            for r in range(N // 8):
                out_ref[N * j + 8 * r : N * j + 8 * (r + 1), :] = jnp.where(
                    row8 == sub8 + 8 * r, 1.0, 0.0
                )

    @pl.when(i == N_MAIN)
    def _tail():
        card = card_full_ref[...]
        flip = flip_ref[...]
        prow = jax.lax.broadcasted_iota(jnp.int32, (TWO_N, B), 0)
        fcv = jnp.where(prow == flip, card, 0)
        fc = jnp.sum(fcv, axis=0, keepdims=True)
        sub = jax.lax.broadcasted_iota(jnp.int32, (N, B), 0)
        valid = valid_ref[...]
        out_ref[0:N, :] = jnp.where(sub == fc, valid, 0.0)
        par = jnp.bitwise_and(t_ref[...], 1)
        sub2 = jax.lax.broadcasted_iota(jnp.int32, (2, B), 0)
        out_ref[N : N + 2, :] = jnp.where(sub2 == par, 1.0, 0.0)


def kernel(card, seen_mask, flipped, flipped_valid, t, W):
    del W
    card32 = card.astype(jnp.int32)
    cardmT = jnp.where(seen_mask, card32, 64).T
    cardT = card32.T
    flipT = flipped.astype(jnp.int32).reshape(1, B)
    validT = flipped_valid.astype(jnp.float32).reshape(1, B)
    tT = t.astype(jnp.int32).reshape(1, B)

    grid = (N_MAIN + 1,)
    out = pl.pallas_call(
        _body,
        grid=grid,
        in_specs=[
            pl.BlockSpec((P_PER, B), lambda i: (jnp.minimum(i, N_MAIN - 1), 0)),
            pl.BlockSpec((TWO_N, B), lambda i: (0, 0)),
            pl.BlockSpec((1, B), lambda i: (0, 0)),
            pl.BlockSpec((1, B), lambda i: (0, 0)),
            pl.BlockSpec((1, B), lambda i: (0, 0)),
        ],
        out_specs=pl.BlockSpec((FB, None, B), lambda i: (i, 0, 0)),
        out_shape=jax.ShapeDtypeStruct((OUT_W, 1, B), jnp.float32),
    )(cardmT, cardT, flipT, validT, tT)
    return jnp.transpose(out, (2, 1, 0))
